# baseline (speedup 1.0000x reference)
import functools

import jax
import jax.numpy as jnp
from jax import lax
from jax.experimental import pallas as pl
from jax.experimental.pallas import tpu as pltpu
from jax.experimental.pallas import tpu_sc as plsc

_LANES = 16


def _make_sc_add(B, T, D, V):
    info = plsc.get_sparse_core_info()
    NC, NS = info.num_cores, info.num_subcores
    NW = NC * NS
    t_per_w = T // NW
    CH = min(8, t_per_w)
    n_chunks = t_per_w // CH
    NB = min(3, n_chunks)
    UNROLL = 8
    n_grp = D // _LANES
    mesh = plsc.VectorSubcoreMesh(core_axis_name="c", subcore_axis_name="s")

    @functools.partial(
        pl.kernel,
        mesh=mesh,
        out_type=jax.ShapeDtypeStruct((B, T, D), jnp.float32),
        scratch_types=[
            pltpu.VMEM((t_per_w // CH, CH), jnp.int32),
            pltpu.VMEM((NB, CH, D), jnp.float32),
            pltpu.VMEM((NB, B, CH, D), jnp.float32),
            pltpu.SemaphoreType.DMA((NB,)),
            pltpu.SemaphoreType.DMA((NB,)),
            pltpu.SemaphoreType.DMA((NB,)),
        ],
    )
    def sc_add(x_hbm, idx_hbm, pos_hbm, out_hbm, idx_all, pos_v, x_v,
               sg, sx, so):
        wid = lax.axis_index("s") * NC + lax.axis_index("c")
        t_base = wid * t_per_w
        pltpu.sync_copy(idx_hbm.at[pl.ds(wid * n_chunks, n_chunks)], idx_all)

        def in_copies(ci, p):
            t0 = t_base + ci * CH
            g = pltpu.make_async_copy(
                pos_hbm.at[idx_all.at[ci]], pos_v.at[p], sg.at[p])
            xc = pltpu.make_async_copy(
                x_hbm.at[:, pl.ds(t0, CH)], x_v.at[p], sx.at[p])
            return g, xc

        def out_copy(ci, p):
            t0 = t_base + ci * CH
            return pltpu.make_async_copy(
                x_v.at[p], out_hbm.at[:, pl.ds(t0, CH)], so.at[p])

        for d in in_copies(0, 0):
            d.start()

        def process(ci, p):
            pn = (p + 1) % NB
            @pl.when(ci + 1 < n_chunks)
            def _():
                @pl.when(ci >= NB - 1)
                def _():
                    out_copy(ci + 1 - NB, pn).wait()
                for d in in_copies(ci + 1, pn):
                    d.start()
            for d in in_copies(ci, p):
                d.wait()
            for r in range(CH):
                def colgrp(jj, c2):
                    for u in range(UNROLL):
                        sl = pl.ds((jj * UNROLL + u) * _LANES, _LANES)
                        pv = pos_v[p, r, sl]
                        for b in range(B):
                            plsc.addupdate(x_v.at[p, b, r, sl], pv)
                    return c2
                lax.fori_loop(0, n_grp // UNROLL, colgrp, 0)
            out_copy(ci, p).start()

        n_main = (n_chunks // NB) * NB

        def chunk(cg, carry):
            for p in range(NB):
                process(cg * NB + p, p)
            return carry

        lax.fori_loop(0, n_chunks // NB, chunk, 0)
        for ci in range(n_main, n_chunks):
            process(ci, ci % NB)
        for k in range(NB):
            ci = n_chunks - NB + k
            out_copy(ci, ci % NB).wait()

    def run(x, positions, pos_embedding):
        return sc_add(x, positions.reshape(T // CH, CH), pos_embedding)

    return run


def kernel(x, offset, pos_embedding):
    B, T, D = x.shape
    V = pos_embedding.shape[0]
    positions = jnp.clip(
        jnp.arange(T, dtype=jnp.int32) + jnp.asarray(offset, jnp.int32).astype(jnp.int32),
        0, V - 1)
    return _make_sc_add(B, T, D, V)(x, positions, pos_embedding)


# speedup vs baseline: 1.0244x; 1.0244x over previous
import functools

import jax
import jax.numpy as jnp
from jax import lax
from jax.experimental import pallas as pl
from jax.experimental.pallas import tpu as pltpu
from jax.experimental.pallas import tpu_sc as plsc

_LANES = 16


def _make_sc_add(B, T, D, V):
    info = plsc.get_sparse_core_info()
    NC, NS = info.num_cores, info.num_subcores
    NW = NC * NS
    t_per_w = T // NW
    CH = min(8, t_per_w)
    n_chunks = t_per_w // CH
    NB = min(2, n_chunks)
    UNROLL = 8
    n_grp = D // _LANES
    mesh = plsc.VectorSubcoreMesh(core_axis_name="c", subcore_axis_name="s")

    @functools.partial(
        pl.kernel,
        mesh=mesh,
        out_type=jax.ShapeDtypeStruct((B, T, D), jnp.float32),
        scratch_types=[
            pltpu.VMEM((t_per_w // CH, CH), jnp.int32),
            pltpu.VMEM((NB, CH, D), jnp.float32),
            pltpu.VMEM((NB, B, CH, D), jnp.float32),
            pltpu.SemaphoreType.DMA((NB,)),
            pltpu.SemaphoreType.DMA((NB,)),
            pltpu.SemaphoreType.DMA((NB,)),
        ],
    )
    def sc_add(x_hbm, idx_hbm, pos_hbm, out_hbm, idx_all, pos_v, x_v,
               sg, sx, so):
        wid = lax.axis_index("s") * NC + lax.axis_index("c")
        t_base = wid * t_per_w
        pltpu.sync_copy(idx_hbm.at[pl.ds(wid * n_chunks, n_chunks)], idx_all)

        def in_copies(ci, p):
            t0 = t_base + ci * CH
            g = pltpu.make_async_copy(
                pos_hbm.at[idx_all.at[ci]], pos_v.at[p], sg.at[p])
            xc = pltpu.make_async_copy(
                x_hbm.at[:, pl.ds(t0, CH)], x_v.at[p], sx.at[p])
            return g, xc

        def out_copy(ci, p):
            t0 = t_base + ci * CH
            return pltpu.make_async_copy(
                x_v.at[p], out_hbm.at[:, pl.ds(t0, CH)], so.at[p])

        for d in in_copies(0, 0):
            d.start()

        def process(ci, p):
            pn = (p + 1) % NB
            @pl.when(ci + 1 < n_chunks)
            def _():
                @pl.when(ci >= NB - 1)
                def _():
                    out_copy(ci + 1 - NB, pn).wait()
                for d in in_copies(ci + 1, pn):
                    d.start()
            for d in in_copies(ci, p):
                d.wait()
            for r in range(CH):
                def colgrp(jj, c2):
                    for u in range(UNROLL):
                        sl = pl.ds((jj * UNROLL + u) * _LANES, _LANES)
                        pv = pos_v[p, r, sl]
                        for b in range(B):
                            plsc.addupdate(x_v.at[p, b, r, sl], pv)
                    return c2
                lax.fori_loop(0, n_grp // UNROLL, colgrp, 0)
            out_copy(ci, p).start()

        n_main = (n_chunks // NB) * NB

        def chunk(cg, carry):
            for p in range(NB):
                process(cg * NB + p, p)
            return carry

        lax.fori_loop(0, n_chunks // NB, chunk, 0)
        for ci in range(n_main, n_chunks):
            process(ci, ci % NB)
        for k in range(NB):
            ci = n_chunks - NB + k
            out_copy(ci, ci % NB).wait()

    def run(x, positions, pos_embedding):
        return sc_add(x, positions.reshape(T // CH, CH), pos_embedding)

    return run


def kernel(x, offset, pos_embedding):
    B, T, D = x.shape
    V = pos_embedding.shape[0]
    positions = jnp.clip(
        jnp.arange(T, dtype=jnp.int32) + jnp.asarray(offset, jnp.int32).astype(jnp.int32),
        0, V - 1)
    return _make_sc_add(B, T, D, V)(x, positions, pos_embedding)
